# Initial kernel scaffold; baseline (speedup 1.0000x reference)
import jax
import jax.numpy as jnp
from jax.experimental import pallas as pl


def kernel(h, edge_index, e, snorm_n, W, b, gamma, beta):
    raise NotImplementedError("write your pallas kernel here")


# speedup vs baseline: 1.8720x; 1.8720x over previous
import dataclasses
import functools

import jax
import jax.numpy as jnp
from jax import lax
from jax.experimental import pallas as pl
from jax.experimental.pallas import tpu as pltpu
from jax.experimental.pallas import tpu_sc as plsc

N = 10000
E = 320000
D = 128
N_PAD = 10240
K_NODES = 320
NUM_TILES = 32
BLK_E = 640
N_BLKS = E // BLK_E
G_ROWS = 32
AVG_D_LOG = 3.4965
BN_EPS = 1e-5
NEG_BIG = -3.0e38
POS_BIG = 3.0e38


def _sc_aggregate(src_i32, dst_i32, h):
    mesh = plsc.VectorSubcoreMesh(core_axis_name="c", subcore_axis_name="s")
    cp = pltpu.CompilerParams()
    if "needs_layout_passes" in pltpu.CompilerParams.__dataclass_fields__:
        cp = dataclasses.replace(cp, needs_layout_passes=False)
    out_types = (
        jax.ShapeDtypeStruct((N_PAD, D), jnp.float32),
        jax.ShapeDtypeStruct((N_PAD, D), jnp.float32),
        jax.ShapeDtypeStruct((N_PAD, D), jnp.float32),
        jax.ShapeDtypeStruct((NUM_TILES, K_NODES), jnp.float32),
    )

    @functools.partial(
        pl.kernel,
        out_type=out_types,
        mesh=mesh,
        compiler_params=cp,
        scratch_types=[
            pltpu.VMEM((K_NODES, D), jnp.float32),
            pltpu.VMEM((K_NODES, D), jnp.float32),
            pltpu.VMEM((K_NODES, D), jnp.float32),
            pltpu.VMEM((K_NODES,), jnp.float32),
            pltpu.VMEM((BLK_E,), jnp.int32),
            pltpu.VMEM((BLK_E,), jnp.int32),
            pltpu.VMEM((BLK_E + 16,), jnp.int32),
            pltpu.VMEM((BLK_E + 16,), jnp.int32),
            pltpu.VMEM((G_ROWS, D), jnp.float32),
        ],
    )
    def agg_kernel(src_hbm, dst_hbm, h_hbm,
                   out_sum, out_max, out_min, out_deg,
                   acc_sum, acc_max, acc_min, deg_v,
                   src_st, dst_st, comp_src, comp_dst, gbuf):
        wid = lax.axis_index("c") * 16 + lax.axis_index("s")
        base = wid * K_NODES
        iota16 = lax.iota(jnp.int32, 16)
        ones16 = jnp.full((16,), 1.0, jnp.float32)

        @pl.loop(0, K_NODES)
        def _(r):
            for c in range(D // 16):
                sl = pl.ds(c * 16, 16)
                acc_sum[r, sl] = jnp.zeros((16,), jnp.float32)
                acc_max[r, sl] = jnp.full((16,), NEG_BIG, jnp.float32)
                acc_min[r, sl] = jnp.full((16,), POS_BIG, jnp.float32)

        @pl.loop(0, K_NODES // 16)
        def _(i):
            deg_v[pl.ds(i * 16, 16)] = jnp.zeros((16,), jnp.float32)

        @pl.loop(0, BLK_E // 16)
        def _(i):
            comp_src[pl.ds(i * 16, 16)] = jnp.zeros((16,), jnp.int32)
            comp_dst[pl.ds(i * 16, 16)] = jnp.zeros((16,), jnp.int32)

        def process_block(blk):
            pltpu.sync_copy(src_hbm.at[pl.ds(blk * BLK_E, BLK_E)], src_st)
            pltpu.sync_copy(dst_hbm.at[pl.ds(blk * BLK_E, BLK_E)], dst_st)

            def scan_body(ch, cnt):
                dv = dst_st[pl.ds(ch * 16, 16)]
                sv = src_st[pl.ds(ch * 16, 16)]
                m = (dv >= base) & (dv < base + K_NODES)
                dloc = jnp.clip(dv - base, 0, K_NODES - 1)
                plsc.addupdate_scatter(deg_v, [dloc], ones16, mask=m)
                plsc.store_compressed(comp_src.at[pl.ds(cnt, 16)], sv, mask=m)
                plsc.store_compressed(comp_dst.at[pl.ds(cnt, 16)], dloc, mask=m)
                return cnt + jnp.sum(m.astype(jnp.int32))

            cnt = lax.fori_loop(0, BLK_E // 16, scan_body, 0)

            def group_body(g, _):
                pltpu.sync_copy(h_hbm.at[comp_src.at[pl.ds(g * G_ROWS, G_ROWS)]],
                                gbuf)
                r = jnp.minimum(cnt - g * G_ROWS, G_ROWS)

                def edge_body(j, _):
                    i = g * G_ROWS + j
                    off = (i >> 4) << 4
                    lane = i & 15
                    dvec = comp_dst[pl.ds(off, 16)]
                    d = jnp.sum(jnp.where(iota16 == lane, dvec, 0))
                    for c in range(D // 16):
                        sl = pl.ds(c * 16, 16)
                        x = gbuf[j, sl]
                        acc_sum[d, sl] = acc_sum[d, sl] + x
                        acc_max[d, sl] = jnp.maximum(acc_max[d, sl], x)
                        acc_min[d, sl] = jnp.minimum(acc_min[d, sl], x)
                    return 0

                lax.fori_loop(0, r, edge_body, 0)
                return 0

            n_groups = (cnt + G_ROWS - 1) // G_ROWS
            lax.fori_loop(0, n_groups, group_body, 0)

        @pl.loop(0, N_BLKS)
        def _(blk):
            process_block(blk)

        pltpu.sync_copy(acc_sum, out_sum.at[pl.ds(base, K_NODES)])
        pltpu.sync_copy(acc_max, out_max.at[pl.ds(base, K_NODES)])
        pltpu.sync_copy(acc_min, out_min.at[pl.ds(base, K_NODES)])
        pltpu.sync_copy(deg_v, out_deg.at[wid])

    return agg_kernel(src_i32, dst_i32, h)


TC_BLK = 1000
TC_NBLK = N // TC_BLK


def _tc_tail(asum, amax, amin, deg, h, snorm, Wr, b2, gamma2, beta2):
    def pre_kernel(asum_r, amax_r, amin_r, deg_r, snorm_r, w_r, b_r,
                   out1_r, stats_r, acc_r):
        step = pl.program_id(0)
        deg = deg_r[...]
        pos = deg > 0.0
        degs = jnp.maximum(deg, 1.0)
        mean = asum_r[...] / degs
        mx = jnp.where(pos, amax_r[...], 0.0)
        mn = jnp.where(pos, amin_r[...], 0.0)
        ld = jnp.log(deg + 1.0)
        a_amp = ld * (1.0 / AVG_D_LOG)
        a_att = AVG_D_LOG / jnp.maximum(ld, 1e-6)

        def mm(x, k):
            return lax.dot_general(x, w_r[k],
                                   (((1,), (0,)), ((), ())),
                                   preferred_element_type=jnp.float32)

        s1 = mm(mean, 0) + mm(mx, 1) + mm(mn, 2)
        s2 = mm(mean, 3) + mm(mx, 4) + mm(mn, 5)
        s3 = mm(mean, 6) + mm(mx, 7) + mm(mn, 8)
        out1 = (s1 + a_amp * s2 + a_att * s3 + b_r[...]) * snorm_r[...]
        out1_r[...] = out1
        part = jnp.concatenate(
            [jnp.sum(out1, axis=0, keepdims=True),
             jnp.sum(out1 * out1, axis=0, keepdims=True)], axis=0)

        @pl.when(step == 0)
        def _():
            acc_r[...] = jnp.zeros_like(acc_r)

        acc_r[...] += part
        stats_r[...] = acc_r[...]

    out1, stats = pl.pallas_call(
        pre_kernel,
        grid=(TC_NBLK,),
        in_specs=[
            pl.BlockSpec((TC_BLK, D), lambda i: (i, 0)),
            pl.BlockSpec((TC_BLK, D), lambda i: (i, 0)),
            pl.BlockSpec((TC_BLK, D), lambda i: (i, 0)),
            pl.BlockSpec((TC_BLK, 1), lambda i: (i, 0)),
            pl.BlockSpec((TC_BLK, 1), lambda i: (i, 0)),
            pl.BlockSpec((9, D, D), lambda i: (0, 0, 0)),
            pl.BlockSpec((1, D), lambda i: (0, 0)),
        ],
        out_specs=[
            pl.BlockSpec((TC_BLK, D), lambda i: (i, 0)),
            pl.BlockSpec((2, D), lambda i: (0, 0)),
        ],
        out_shape=[
            jax.ShapeDtypeStruct((N, D), jnp.float32),
            jax.ShapeDtypeStruct((2, D), jnp.float32),
        ],
        scratch_shapes=[pltpu.VMEM((2, D), jnp.float32)],
    )(asum, amax, amin, deg, snorm, Wr, b2)

    def post_kernel(out1_r, h_r, stats_r, gamma_r, beta_r, out_r):
        mu = stats_r[0:1, :] * (1.0 / N)
        var = stats_r[1:2, :] * (1.0 / N) - mu * mu
        y = ((out1_r[...] - mu) * lax.rsqrt(var + BN_EPS) * gamma_r[...]
             + beta_r[...])
        out_r[...] = jnp.maximum(y, 0.0) + h_r[...]

    return pl.pallas_call(
        post_kernel,
        grid=(TC_NBLK,),
        in_specs=[
            pl.BlockSpec((TC_BLK, D), lambda i: (i, 0)),
            pl.BlockSpec((TC_BLK, D), lambda i: (i, 0)),
            pl.BlockSpec((2, D), lambda i: (0, 0)),
            pl.BlockSpec((1, D), lambda i: (0, 0)),
            pl.BlockSpec((1, D), lambda i: (0, 0)),
        ],
        out_specs=pl.BlockSpec((TC_BLK, D), lambda i: (i, 0)),
        out_shape=jax.ShapeDtypeStruct((N, D), jnp.float32),
    )(out1, h, stats, gamma2, beta2)


def kernel(h, edge_index, e, snorm_n, W, b, gamma, beta):
    del e
    src = edge_index[0].astype(jnp.int32)
    dst = edge_index[1].astype(jnp.int32)
    asum, amax, amin, deg = _sc_aggregate(src, dst, h)
    asum = asum[:N]
    amax = amax[:N]
    amin = amin[:N]
    deg = deg.reshape(N_PAD, 1)[:N]
    Wr = W.reshape(9, D, D)
    out = _tc_tail(asum, amax, amin, deg, h, snorm_n,
                   Wr, b.reshape(1, D), gamma.reshape(1, D), beta.reshape(1, D))
    return out
